# baseline (speedup 1.0000x reference)
import jax
import jax.numpy as jnp
from jax import lax
from jax.experimental import pallas as pl

N = 1024
IN = 256
OUT = 128
NH = 4
HD = OUT // NH


def _sem_kernel(x_ref, adj_ref, wlin_ref, wlay_ref, watt_ref, out_ref):
    x = x_ref[...]
    h = lax.dot_general(x, wlin_ref[...], (((1,), (1,)), ((), ())),
                        preferred_element_type=jnp.float32)
    watt = watt_ref[...].reshape(NH, 2 * OUT)
    s = lax.dot_general(h, watt[:, :OUT], (((1,), (1,)), ((), ())),
                        preferred_element_type=jnp.float32)
    t = lax.dot_general(watt[:, OUT:], h, (((1,), (1,)), ((), ())),
                        preferred_element_type=jnp.float32)
    u = jnp.exp(-jnp.clip(s, -30.0, 30.0))
    v = jnp.exp(-jnp.clip(t, -30.0, 30.0))
    wlay = wlay_ref[...].reshape(OUT, IN)
    xw = lax.dot_general(x, wlay, (((1,), (1,)), ((), ())),
                         preferred_element_type=jnp.float32)
    xwb = xw.astype(jnp.bfloat16)

    adjm = adj_ref[...]
    zero = (adjm == 0.0)
    nzeros = jnp.sum(jnp.where(zero, 1.0, 0.0))
    nzb = jnp.where(zero, 0.0, 1.0).astype(jnp.bfloat16)

    for k in range(NH):
        uk = u[:, k:k + 1]
        vk = v[k:k + 1, :]
        akb = (1.0 / (1.0 + uk * vk)).astype(jnp.bfloat16) * nzb
        ok = lax.dot_general(akb, xwb[:, k * HD:(k + 1) * HD],
                             (((1,), (0,)), ((), ())),
                             preferred_element_type=jnp.float32)
        out_ref[:, k * HD:(k + 1) * HD] = ok

    for k in range(NH):
        e00 = 1.0 / (1.0 + u[0:1, k:k + 1] * v[k:k + 1, 0:1])
        cs = slice(k * HD, (k + 1) * HD)
        out_ref[0:1, cs] = out_ref[0:1, cs] + (nzeros * e00) * xw[0:1, cs]


@jax.jit
def kernel(x, adj, W_lin, W_layers, W_atts, W_c1, W_c2):
    del W_c1, W_c2
    return pl.pallas_call(
        _sem_kernel,
        out_shape=jax.ShapeDtypeStruct((N, OUT), jnp.float32),
    )(x, adj, W_lin, W_layers, W_atts)


# speedup vs baseline: 1.0313x; 1.0313x over previous
import jax
import jax.numpy as jnp
from jax import lax
from jax.experimental import pallas as pl

N = 1024
IN = 256
OUT = 128
NH = 4
HD = OUT // NH


def _sem_kernel(x_ref, adj_ref, wlin_ref, wlay_ref, watt_ref, out_ref):
    x = x_ref[...]
    h = lax.dot_general(x, wlin_ref[...], (((1,), (1,)), ((), ())),
                        preferred_element_type=jnp.float32)
    watt = watt_ref[...].reshape(NH, 2 * OUT)
    s = lax.dot_general(h, watt[:, :OUT], (((1,), (1,)), ((), ())),
                        preferred_element_type=jnp.float32)
    t = lax.dot_general(watt[:, OUT:], h, (((1,), (1,)), ((), ())),
                        preferred_element_type=jnp.float32)
    u = jnp.exp(-jnp.clip(s, -30.0, 30.0))
    v = jnp.exp(-jnp.clip(t, -30.0, 30.0))
    wlay = wlay_ref[...].reshape(OUT, IN)
    xw = lax.dot_general(x, wlay, (((1,), (1,)), ((), ())),
                         preferred_element_type=jnp.float32)
    xwb = xw.astype(jnp.bfloat16)

    adjm = adj_ref[...]
    zero = (adjm == 0.0)
    nzeros = jnp.sum(jnp.where(zero, 1.0, 0.0))
    onez = jnp.where(zero, 2e38, 1.0)

    for k in range(NH):
        uk = u[:, k:k + 1]
        vk = v[k:k + 1, :]
        akb = (1.0 / (uk * vk + onez)).astype(jnp.bfloat16)
        ok = lax.dot_general(akb, xwb[:, k * HD:(k + 1) * HD],
                             (((1,), (0,)), ((), ())),
                             preferred_element_type=jnp.float32)
        out_ref[:, k * HD:(k + 1) * HD] = ok

    for k in range(NH):
        e00 = 1.0 / (1.0 + u[0:1, k:k + 1] * v[k:k + 1, 0:1])
        cs = slice(k * HD, (k + 1) * HD)
        out_ref[0:1, cs] = out_ref[0:1, cs] + (nzeros * e00) * xw[0:1, cs]


@jax.jit
def kernel(x, adj, W_lin, W_layers, W_atts, W_c1, W_c2):
    del W_c1, W_c2
    return pl.pallas_call(
        _sem_kernel,
        out_shape=jax.ShapeDtypeStruct((N, OUT), jnp.float32),
    )(x, adj, W_lin, W_layers, W_atts)
